# Initial kernel scaffold; baseline (speedup 1.0000x reference)
import jax
import jax.numpy as jnp
from jax.experimental import pallas as pl


def kernel(x, edge_index, W0, b0, W1, b1, W2, b2):
    raise NotImplementedError("write your pallas kernel here")


# speedup vs baseline: 13.0238x; 13.0238x over previous
import functools

import jax
import jax.numpy as jnp
from jax import lax
from jax.experimental import pallas as pl
from jax.experimental.pallas import tpu as pltpu
from jax.experimental.pallas import tpu_sc as plsc

NC = 2
NS = 16
NW = NC * NS
CHUNK = 128


def _sc_mesh():
    return plsc.VectorSubcoreMesh(core_axis_name="c", subcore_axis_name="s")


def _make_deg_kernel(npad, cpt, rpt):
    @functools.partial(
        pl.kernel,
        out_type=jax.ShapeDtypeStruct((NC, npad), jnp.float32),
        mesh=_sc_mesh(),
        scratch_types=[
            pltpu.VMEM((cpt, CHUNK), jnp.int32),
            pltpu.VMEM((CHUNK,), jnp.float32),
            pltpu.VMEM((rpt,), jnp.float32),
            pltpu.VMEM_SHARED((npad,), jnp.float32),
        ],
    )
    def deg_kernel(dst_hbm, h_hbm, idx_v, ones_v, stage_v, acc):
        c = lax.axis_index("c")
        s = lax.axis_index("s")
        tid = c * NS + s
        pltpu.sync_copy(dst_hbm.at[tid], idx_v)
        for m in range(CHUNK // 16):
            ones_v[pl.ds(16 * m, 16)] = jnp.ones((16,), jnp.float32)

        @pl.loop(0, rpt, step=16)
        def _(r):
            stage_v[pl.ds(r, 16)] = jnp.zeros((16,), jnp.float32)

        pltpu.sync_copy(stage_v, acc.at[pl.ds(s * rpt, rpt)])
        plsc.subcore_barrier()

        @pl.loop(0, cpt)
        def _(j):
            pltpu.sync_copy(ones_v, acc.at[idx_v.at[j]], add=True)

        plsc.subcore_barrier()
        pltpu.sync_copy(acc.at[pl.ds(s * rpt, rpt)], stage_v)
        pltpu.sync_copy(stage_v, h_hbm.at[c].at[pl.ds(s * rpt, rpt)])

    return deg_kernel


def _make_agg_kernel(npad, cpt, rpt, d):
    @functools.partial(
        pl.kernel,
        out_type=jax.ShapeDtypeStruct((NC, npad, d), jnp.float32),
        mesh=_sc_mesh(),
        scratch_types=[
            pltpu.VMEM((4, CHUNK), jnp.int32),
            pltpu.VMEM((cpt, CHUNK), jnp.int32),
            pltpu.VMEM((2, CHUNK, d), jnp.float32),
            pltpu.VMEM((32, d), jnp.float32),
            pltpu.VMEM_SHARED((npad, d), jnp.float32),
            pltpu.SemaphoreType.DMA,
            pltpu.SemaphoreType.DMA,
            pltpu.SemaphoreType.DMA,
            pltpu.SemaphoreType.DMA,
        ],
    )
    def agg_kernel(src_hbm, dst_hbm, xs_hbm, p_hbm,
                   ring, didx, buf, wb, acc, sem0, sem1, semi0, semi1):
        c = lax.axis_index("c")
        s = lax.axis_index("s")
        tid = c * NS + s
        sems = (sem0, sem1)
        semis = (semi0, semi1)
        pltpu.sync_copy(dst_hbm.at[tid], didx)

        for r in range(32):
            for m in range(d // 16):
                wb[r, pl.ds(16 * m, 16)] = jnp.zeros((16,), jnp.float32)

        @pl.loop(0, rpt, step=32)
        def _(r):
            pltpu.sync_copy(wb, acc.at[pl.ds(s * rpt + r, 32)])

        plsc.subcore_barrier()

        def prefetch_idx(jj, k, b):
            pltpu.async_copy(src_hbm.at[tid].at[jj], ring.at[k], semis[b])

        def wait_idx(b):
            pltpu.make_async_copy(src_hbm.at[tid].at[0], ring.at[0], semis[b]).wait()

        def issue_gather(k, b):
            pltpu.async_copy(xs_hbm.at[ring.at[k]], buf.at[b], sems[b])

        def wait_gather(b):
            pltpu.make_async_copy(xs_hbm.at[ring.at[0]], buf.at[b], sems[b]).wait()

        def scatter(jj, b):
            pltpu.sync_copy(buf.at[b], acc.at[didx.at[jj]], add=True)

        pltpu.sync_copy(src_hbm.at[tid].at[0], ring.at[0])
        pltpu.sync_copy(src_hbm.at[tid].at[1], ring.at[1])
        prefetch_idx(2, 2, 0)
        prefetch_idx(3, 3, 1)
        issue_gather(0, 0)
        issue_gather(1, 1)

        @pl.loop(0, cpt - 4, step=4)
        def _(g):
            for k in range(4):
                b = k % 2
                wait_gather(b)
                scatter(g + k, b)
                wait_idx(b)
                issue_gather((k + 2) % 4, b)
                prefetch_idx(g + k + 4, k, b)

        for k in range(4):
            b = k % 2
            j = cpt - 4 + k
            wait_gather(b)
            scatter(j, b)
            if j + 2 < cpt:
                wait_idx(b)
                issue_gather((k + 2) % 4, b)

        plsc.subcore_barrier()

        @pl.loop(0, rpt, step=32)
        def _(r):
            pltpu.sync_copy(acc.at[pl.ds(s * rpt + r, 32)], wb)
            pltpu.sync_copy(wb, p_hbm.at[c].at[pl.ds(s * rpt + r, 32)])

    return agg_kernel


def _scale_body(x_ref, dv_ref, o_ref):
    o_ref[...] = x_ref[...] * dv_ref[...]


def _layer1_body(p0_ref, p1_ref, xs_ref, dv_ref, w_ref, b_ref, o_ref):
    g = (p0_ref[...] + p1_ref[...] + xs_ref[...]) * dv_ref[...]
    h = jnp.dot(g, w_ref[...], preferred_element_type=jnp.float32) + b_ref[...]
    o_ref[...] = jnp.maximum(h, 0.0) * dv_ref[...]


def _layer23_body(q0_ref, q1_ref, hs_ref, dv_ref, w1_ref, b1_ref,
                  w2_ref, b2_ref, o1_ref, o2_ref):
    g = (q0_ref[...] + q1_ref[...] + hs_ref[...]) * dv_ref[...]
    o1_ref[...] = jnp.dot(g, w1_ref[...], preferred_element_type=jnp.float32) + b1_ref[...]
    o2_ref[...] = jnp.dot(g, w2_ref[...], preferred_element_type=jnp.float32) + b2_ref[...]


def kernel(x, edge_index, W0, b0, W1, b1, W2, b2):
    n, d = x.shape
    e = edge_index.shape[1]

    npad_deg = ((n + 1 + 128 * NS - 1) // (128 * NS)) * (128 * NS)
    npad = ((n + 1 + 32 * NS - 1) // (32 * NS)) * (32 * NS)
    cpt = -(-e // (NW * CHUNK))
    cpt = ((cpt + 3) // 4) * 4
    epad = NW * cpt * CHUNK

    src = edge_index[0].astype(jnp.int32)
    dst = edge_index[1].astype(jnp.int32)
    pad = epad - e
    src_p = jnp.concatenate([src, jnp.zeros((pad,), jnp.int32)]).reshape(NW, cpt, CHUNK)
    dst_pad = n + (jnp.arange(pad, dtype=jnp.int32) % (npad - n))
    dst_p = jnp.concatenate([dst, dst_pad]).reshape(NW, cpt, CHUNK)

    hist = _make_deg_kernel(npad_deg, cpt, npad_deg // NS)(dst_p)
    deg = hist[0, :n] + hist[1, :n] + 1.0
    dv = lax.rsqrt(deg)[:, None]

    agg = _make_agg_kernel(npad, cpt, npad // NS, d)

    grid_r = 1000
    grid = (n // grid_r,)
    row_spec = pl.BlockSpec((grid_r, d), lambda i: (i, 0))
    col_spec = pl.BlockSpec((grid_r, 1), lambda i: (i, 0))
    w_spec = pl.BlockSpec((d, d), lambda i: (0, 0))
    b_spec = pl.BlockSpec((1, d), lambda i: (0, 0))
    out_nd = jax.ShapeDtypeStruct((n, d), jnp.float32)

    xs = pl.pallas_call(
        _scale_body,
        grid=grid,
        in_specs=[row_spec, col_spec],
        out_specs=row_spec,
        out_shape=out_nd,
    )(x, dv)

    p = agg(src_p, dst_p, xs)

    hs0 = pl.pallas_call(
        _layer1_body,
        grid=grid,
        in_specs=[row_spec, row_spec, row_spec, col_spec, w_spec, b_spec],
        out_specs=row_spec,
        out_shape=out_nd,
    )(p[0, :n], p[1, :n], xs, dv, W0, b0.reshape(1, d))

    q = agg(src_p, dst_p, hs0)

    x_, x2 = pl.pallas_call(
        _layer23_body,
        grid=grid,
        in_specs=[row_spec, row_spec, row_spec, col_spec,
                  w_spec, b_spec, w_spec, b_spec],
        out_specs=(row_spec, row_spec),
        out_shape=(out_nd, out_nd),
    )(q[0, :n], q[1, :n], hs0, dv, W1, b1.reshape(1, d), W2, b2.reshape(1, d))

    return (x_, x2)


# speedup vs baseline: 13.4180x; 1.0303x over previous
import functools

import jax
import jax.numpy as jnp
from jax import lax
from jax.experimental import pallas as pl
from jax.experimental.pallas import tpu as pltpu
from jax.experimental.pallas import tpu_sc as plsc

NC = 2
NS = 16
NW = NC * NS
CHUNK = 128
SPLIT_NUM, SPLIT_DEN = 3, 4


def _sc_mesh():
    return plsc.VectorSubcoreMesh(core_axis_name="c", subcore_axis_name="s")


def _core_chunks(e):
    pair = -(-e // (NS * CHUNK))
    cpt0 = max(8, ((pair * SPLIT_NUM // SPLIT_DEN + 3) // 4) * 4)
    cpt1 = max(8, ((pair - cpt0 + 3) // 4) * 4)
    return cpt0, cpt1


def _make_deg_kernel(npad, cpt0, cpt1, rpt):
    cptmax = max(cpt0, cpt1)

    @functools.partial(
        pl.kernel,
        out_type=jax.ShapeDtypeStruct((NC, npad), jnp.float32),
        mesh=_sc_mesh(),
        scratch_types=[
            pltpu.VMEM((cptmax, CHUNK), jnp.int32),
            pltpu.VMEM((CHUNK,), jnp.float32),
            pltpu.VMEM((rpt,), jnp.float32),
            pltpu.VMEM_SHARED((npad,), jnp.float32),
        ],
    )
    def deg_kernel(dst_hbm, h_hbm, idx_v, ones_v, stage_v, acc):
        c = lax.axis_index("c")
        s = lax.axis_index("s")
        tid = c * NS + s
        pltpu.sync_copy(dst_hbm.at[tid], idx_v)
        for m in range(CHUNK // 16):
            ones_v[pl.ds(16 * m, 16)] = jnp.ones((16,), jnp.float32)

        @pl.loop(0, rpt, step=16)
        def _(r):
            stage_v[pl.ds(r, 16)] = jnp.zeros((16,), jnp.float32)

        pltpu.sync_copy(stage_v, acc.at[pl.ds(s * rpt, rpt)])
        plsc.subcore_barrier()

        def scatter_ones(cpt):
            @pl.loop(0, cpt)
            def _(j):
                pltpu.sync_copy(ones_v, acc.at[idx_v.at[j]], add=True)

        @pl.when(c == 0)
        def _():
            scatter_ones(cpt0)

        @pl.when(c == 1)
        def _():
            scatter_ones(cpt1)

        plsc.subcore_barrier()
        pltpu.sync_copy(acc.at[pl.ds(s * rpt, rpt)], stage_v)
        pltpu.sync_copy(stage_v, h_hbm.at[c].at[pl.ds(s * rpt, rpt)])

    return deg_kernel


def _make_agg_kernel(npad, cpt0, cpt1, rpt, d):
    cptmax = max(cpt0, cpt1)
    assert rpt % CHUNK == 0 and min(cpt0, cpt1) >= 8

    @functools.partial(
        pl.kernel,
        out_type=jax.ShapeDtypeStruct((NC, npad, d), jnp.float32),
        mesh=_sc_mesh(),
        scratch_types=[
            pltpu.VMEM((4, CHUNK), jnp.int32),
            pltpu.VMEM((cptmax, CHUNK), jnp.int32),
            pltpu.VMEM((2, CHUNK, d), jnp.float32),
            pltpu.VMEM_SHARED((npad, d), jnp.float32),
            pltpu.SemaphoreType.DMA,
            pltpu.SemaphoreType.DMA,
            pltpu.SemaphoreType.DMA,
            pltpu.SemaphoreType.DMA,
        ],
    )
    def agg_kernel(src_hbm, dst_hbm, xs_hbm, p_hbm,
                   ring, didx, buf, acc, sem0, sem1, semi0, semi1):
        c = lax.axis_index("c")
        s = lax.axis_index("s")
        tid = c * NS + s
        sems = (sem0, sem1)
        semis = (semi0, semi1)
        pltpu.sync_copy(dst_hbm.at[tid], didx)

        for r in range(CHUNK):
            for m in range(d // 16):
                buf[0, r, pl.ds(16 * m, 16)] = jnp.zeros((16,), jnp.float32)
        for r in range(0, rpt, CHUNK):
            pltpu.async_copy(buf.at[0], acc.at[pl.ds(s * rpt + r, CHUNK)], sem0)
        for r in range(0, rpt, CHUNK):
            pltpu.make_async_copy(buf.at[0], acc.at[pl.ds(0, CHUNK)], sem0).wait()

        plsc.subcore_barrier()

        def prefetch_idx(jj, k, b):
            pltpu.async_copy(src_hbm.at[tid].at[jj], ring.at[k], semis[b])

        def wait_idx(b):
            pltpu.make_async_copy(src_hbm.at[tid].at[0], ring.at[0], semis[b]).wait()

        def issue_gather(k, b):
            pltpu.async_copy(xs_hbm.at[ring.at[k]], buf.at[b], sems[b])

        def wait_gather(b):
            pltpu.make_async_copy(xs_hbm.at[ring.at[0]], buf.at[b], sems[b]).wait()

        def scatter(jj, b):
            pltpu.sync_copy(buf.at[b], acc.at[didx.at[jj]], add=True)

        def run_pipeline(cpt):
            pltpu.sync_copy(src_hbm.at[tid].at[0], ring.at[0])
            pltpu.sync_copy(src_hbm.at[tid].at[1], ring.at[1])
            prefetch_idx(2, 2, 0)
            prefetch_idx(3, 3, 1)
            issue_gather(0, 0)
            issue_gather(1, 1)

            @pl.loop(0, cpt - 4, step=4)
            def _(g):
                for k in range(4):
                    b = k % 2
                    wait_gather(b)
                    scatter(g + k, b)
                    wait_idx(b)
                    issue_gather((k + 2) % 4, b)
                    prefetch_idx(g + k + 4, k, b)

            for k in range(4):
                b = k % 2
                j = cpt - 4 + k
                wait_gather(b)
                scatter(j, b)
                if j + 2 < cpt:
                    wait_idx(b)
                    issue_gather((k + 2) % 4, b)

        @pl.when(c == 0)
        def _():
            run_pipeline(cpt0)

        @pl.when(c == 1)
        def _():
            run_pipeline(cpt1)

        plsc.subcore_barrier()

        nwb = rpt // CHUNK
        pltpu.async_copy(acc.at[pl.ds(s * rpt, CHUNK)], buf.at[0], sem0)
        for i in range(nwb):
            b = i % 2
            pltpu.make_async_copy(acc.at[pl.ds(0, CHUNK)], buf.at[b], sems[b]).wait()
            if i + 1 < nwb:
                pltpu.async_copy(acc.at[pl.ds(s * rpt + (i + 1) * CHUNK, CHUNK)],
                                 buf.at[1 - b], sems[1 - b])
            pltpu.sync_copy(buf.at[b],
                            p_hbm.at[c].at[pl.ds(s * rpt + i * CHUNK, CHUNK)])

    return agg_kernel


def _partition_edges(src, dst, n, npad, cpt0, cpt1):
    e = src.shape[0]
    cptmax = max(cpt0, cpt1)
    srcs, dsts = [], []
    off = 0
    for cpt in (cpt0, cpt1):
        cap = NS * cpt * CHUNK
        take = min(max(e - off, 0), cap)
        padlen = cap - take
        s_part = jnp.concatenate(
            [src[off:off + take], jnp.zeros((padlen,), jnp.int32)])
        d_part = jnp.concatenate(
            [dst[off:off + take],
             n + (jnp.arange(padlen, dtype=jnp.int32) % (npad - n))])
        s_part = s_part.reshape(NS, cpt, CHUNK)
        d_part = d_part.reshape(NS, cpt, CHUNK)
        if cpt < cptmax:
            padc = ((0, 0), (0, cptmax - cpt), (0, 0))
            s_part = jnp.pad(s_part, padc)
            d_part = jnp.pad(d_part, padc)
        srcs.append(s_part)
        dsts.append(d_part)
        off += take
    return jnp.concatenate(srcs), jnp.concatenate(dsts)


def _scale_body(x_ref, dv_ref, o_ref):
    o_ref[...] = x_ref[...] * dv_ref[...]


def _layer1_body(p0_ref, p1_ref, xs_ref, dv_ref, w_ref, b_ref, o_ref):
    g = (p0_ref[...] + p1_ref[...] + xs_ref[...]) * dv_ref[...]
    h = jnp.dot(g, w_ref[...], preferred_element_type=jnp.float32) + b_ref[...]
    o_ref[...] = jnp.maximum(h, 0.0) * dv_ref[...]


def _layer23_body(q0_ref, q1_ref, hs_ref, dv_ref, w1_ref, b1_ref,
                  w2_ref, b2_ref, o1_ref, o2_ref):
    g = (q0_ref[...] + q1_ref[...] + hs_ref[...]) * dv_ref[...]
    o1_ref[...] = jnp.dot(g, w1_ref[...], preferred_element_type=jnp.float32) + b1_ref[...]
    o2_ref[...] = jnp.dot(g, w2_ref[...], preferred_element_type=jnp.float32) + b2_ref[...]


def kernel(x, edge_index, W0, b0, W1, b1, W2, b2):
    n, d = x.shape
    e = edge_index.shape[1]

    npad = ((n + 1 + 128 * NS - 1) // (128 * NS)) * (128 * NS)
    cpt0, cpt1 = _core_chunks(e)

    src = edge_index[0].astype(jnp.int32)
    dst = edge_index[1].astype(jnp.int32)
    src_p, dst_p = _partition_edges(src, dst, n, npad, cpt0, cpt1)

    hist = _make_deg_kernel(npad, cpt0, cpt1, npad // NS)(dst_p)
    deg = hist[0, :n] + hist[1, :n] + 1.0
    dv = lax.rsqrt(deg)[:, None]

    agg = _make_agg_kernel(npad, cpt0, cpt1, npad // NS, d)

    grid_r = 1000
    grid = (n // grid_r,)
    row_spec = pl.BlockSpec((grid_r, d), lambda i: (i, 0))
    col_spec = pl.BlockSpec((grid_r, 1), lambda i: (i, 0))
    w_spec = pl.BlockSpec((d, d), lambda i: (0, 0))
    b_spec = pl.BlockSpec((1, d), lambda i: (0, 0))
    out_nd = jax.ShapeDtypeStruct((n, d), jnp.float32)

    xs = pl.pallas_call(
        _scale_body,
        grid=grid,
        in_specs=[row_spec, col_spec],
        out_specs=row_spec,
        out_shape=out_nd,
    )(x, dv)

    p = agg(src_p, dst_p, xs)

    hs0 = pl.pallas_call(
        _layer1_body,
        grid=grid,
        in_specs=[row_spec, row_spec, row_spec, col_spec, w_spec, b_spec],
        out_specs=row_spec,
        out_shape=out_nd,
    )(p[0, :n], p[1, :n], xs, dv, W0, b0.reshape(1, d))

    q = agg(src_p, dst_p, hs0)

    x_, x2 = pl.pallas_call(
        _layer23_body,
        grid=grid,
        in_specs=[row_spec, row_spec, row_spec, col_spec,
                  w_spec, b_spec, w_spec, b_spec],
        out_specs=(row_spec, row_spec),
        out_shape=(out_nd, out_nd),
    )(q[0, :n], q[1, :n], hs0, dv, W1, b1.reshape(1, d), W2, b2.reshape(1, d))

    return (x_, x2)
